# baseline (speedup 1.0000x reference)
import functools

import jax
import jax.numpy as jnp
from jax import lax
from jax.experimental import pallas as pl
from jax.experimental.pallas import tpu as pltpu
from jax.experimental.pallas import tpu_sc as plsc

_GROUP = 128


@functools.lru_cache(maxsize=None)
def _build(batch, hist, vocab, dim):
    info = plsc.get_sparse_core_info()
    nc, ns = info.num_cores, info.num_subcores
    nw = nc * ns
    total = batch * hist
    per_w = total // nw
    ngroups = per_w // _GROUP
    assert per_w * nw == total and ngroups * _GROUP == per_w

    mesh = plsc.VectorSubcoreMesh(core_axis_name="c", subcore_axis_name="s")

    K = 5
    nsuper = ngroups // K
    assert nsuper * K == ngroups and nsuper % 2 == 0
    rows_per_super = K * _GROUP

    @functools.partial(
        pl.kernel,
        mesh=mesh,
        compiler_params=pltpu.CompilerParams(use_tc_tiling_on_sc=False),
        out_type=jax.ShapeDtypeStruct((total, dim), jnp.float32),
        scratch_types=[
            pltpu.VMEM((per_w,), jnp.int32),
            pltpu.VMEM((rows_per_super, dim), jnp.float32),
            pltpu.VMEM((rows_per_super, dim), jnp.float32),
            pltpu.SemaphoreType.DMA,
            pltpu.SemaphoreType.DMA,
            pltpu.SemaphoreType.DMA,
            pltpu.SemaphoreType.DMA,
        ],
    )
    def gather_kernel(
        idx_hbm, table_hbm, out_hbm, idx_v, sb0, sb1, gsem0, gsem1, osem0, osem1
    ):
        wid = lax.axis_index("s") * nc + lax.axis_index("c")
        base = wid * per_w
        pltpu.sync_copy(idx_hbm.at[pl.ds(base, per_w)], idx_v)

        sbs = (sb0, sb1)
        gsems = (gsem0, gsem1)
        osems = (osem0, osem1)

        def fire(s, p):
            for j in range(K):
                pltpu.async_copy(
                    table_hbm.at[idx_v.at[pl.ds((s * K + j) * _GROUP, _GROUP)]],
                    sbs[p].at[pl.ds(j * _GROUP, _GROUP)],
                    gsems[p],
                )

        def drain_gathers(p):
            pltpu.make_async_copy(
                out_hbm.at[pl.ds(0, rows_per_super)], sbs[p], gsems[p]
            ).wait()

        def issue_out(s, p):
            pltpu.async_copy(
                sbs[p],
                out_hbm.at[pl.ds(base + s * rows_per_super, rows_per_super)],
                osems[p],
            )

        def wait_out(p):
            pltpu.make_async_copy(
                sbs[p], out_hbm.at[pl.ds(0, rows_per_super)], osems[p]
            ).wait()

        def body(i, carry):
            for p in (0, 1):
                s = 2 * i + p

                @pl.when(s >= 1)
                def _():
                    drain_gathers(1 - p)
                    issue_out(s - 1, 1 - p)

                @pl.when(s >= 2)
                def _():
                    wait_out(p)

                fire(s, p)
            return carry

        lax.fori_loop(0, nsuper // 2, body, 0)

        last_p = (nsuper - 1) % 2
        drain_gathers(last_p)
        issue_out(nsuper - 1, last_p)
        wait_out(1 - last_p)
        wait_out(last_p)

    return gather_kernel


@functools.lru_cache(maxsize=None)
def _build_detile(vocab, dim, blk):
    rows_out = vocab * dim // 128
    br = blk * dim // 128

    packs = 128 // dim

    def detile_kernel(src_ref, dst_ref):
        y = src_ref[...].T.reshape(br, packs, dim)
        dst_ref[...] = jnp.concatenate(
            [y[:, j] for j in range(packs)], axis=1
        )

    return pl.pallas_call(
        detile_kernel,
        grid=(pl.cdiv(vocab, blk),),
        in_specs=[pl.BlockSpec((dim, blk), lambda g: (0, g))],
        out_specs=pl.BlockSpec((br, 128), lambda g: (g, 0)),
        out_shape=jax.ShapeDtypeStruct((rows_out, 128), jnp.float32),
    )


def kernel(indices, table):
    batch, hist = indices.shape
    vocab, dim = table.shape
    total = batch * hist
    zero = lax.optimization_barrier(jnp.zeros((), jnp.int32))
    one = lax.optimization_barrier(jnp.ones((), jnp.float32))
    idx1 = lax.optimization_barrier(
        (indices.astype(jnp.int32) + zero).reshape(total)
    )
    det = _build_detile(vocab, dim, 12800)(table.T)
    del one
    out = _build(batch, hist, vocab, dim)(idx1, det.reshape(vocab, dim))
    return out.reshape(batch, hist, dim)


# speedup vs baseline: 1.2328x; 1.2328x over previous
import functools

import jax
import jax.numpy as jnp
from jax import lax
from jax.experimental import pallas as pl
from jax.experimental.pallas import tpu as pltpu
from jax.experimental.pallas import tpu_sc as plsc

_GROUP = 128


@functools.lru_cache(maxsize=None)
def _build(batch, hist, vocab, dim):
    info = plsc.get_sparse_core_info()
    nc, ns = info.num_cores, info.num_subcores
    nw = nc * ns
    total = batch * hist
    per_w = total // nw
    ngroups = per_w // _GROUP
    assert per_w * nw == total and ngroups * _GROUP == per_w

    mesh = plsc.VectorSubcoreMesh(core_axis_name="c", subcore_axis_name="s")

    K = 5
    nsuper = ngroups // K
    assert nsuper * K == ngroups and nsuper % 2 == 0
    rows_per_super = K * _GROUP

    @functools.partial(
        pl.kernel,
        mesh=mesh,
        compiler_params=pltpu.CompilerParams(use_tc_tiling_on_sc=False),
        out_type=jax.ShapeDtypeStruct((total, dim), jnp.float32),
        scratch_types=[
            pltpu.VMEM((per_w,), jnp.int32),
            pltpu.VMEM((rows_per_super, dim), jnp.float32),
            pltpu.VMEM((rows_per_super, dim), jnp.float32),
            pltpu.SemaphoreType.DMA,
            pltpu.SemaphoreType.DMA,
            pltpu.SemaphoreType.DMA,
            pltpu.SemaphoreType.DMA,
        ],
    )
    def gather_kernel(
        idx_hbm, table_hbm, out_hbm, idx_v, sb0, sb1, gsem0, gsem1, osem0, osem1
    ):
        wid = lax.axis_index("s") * nc + lax.axis_index("c")
        base = wid * per_w
        pltpu.sync_copy(idx_hbm.at[pl.ds(base, per_w)], idx_v)

        sbs = (sb0, sb1)
        gsems = (gsem0, gsem1)
        osems = (osem0, osem1)

        def fire(s, p):
            for j in range(K):
                pltpu.async_copy(
                    table_hbm.at[idx_v.at[pl.ds((s * K + j) * _GROUP, _GROUP)]],
                    sbs[p].at[pl.ds(j * _GROUP, _GROUP)],
                    gsems[p],
                )

        def drain_gathers(p):
            pltpu.make_async_copy(
                out_hbm.at[pl.ds(0, rows_per_super)], sbs[p], gsems[p]
            ).wait()

        def issue_out(s, p):
            pltpu.async_copy(
                sbs[p],
                out_hbm.at[pl.ds(base + s * rows_per_super, rows_per_super)],
                osems[p],
            )

        def wait_out(p):
            pltpu.make_async_copy(
                sbs[p], out_hbm.at[pl.ds(0, rows_per_super)], osems[p]
            ).wait()

        def body(i, carry):
            for p in (0, 1):
                s = 2 * i + p

                @pl.when(s >= 1)
                def _():
                    drain_gathers(1 - p)
                    issue_out(s - 1, 1 - p)

                @pl.when(s >= 2)
                def _():
                    wait_out(p)

                fire(s, p)
            return carry

        lax.fori_loop(0, nsuper // 2, body, 0)

        last_p = (nsuper - 1) % 2
        drain_gathers(last_p)
        issue_out(nsuper - 1, last_p)
        wait_out(1 - last_p)
        wait_out(last_p)

    return gather_kernel


@functools.lru_cache(maxsize=None)
def _build_detile(vocab, dim, blk):
    rows_out = vocab * dim // 128
    br = blk * dim // 128

    packs = 128 // dim

    def detile_kernel(src_ref, dst_ref):
        y = src_ref[...].T.reshape(br, packs, dim)
        dst_ref[...] = jnp.concatenate(
            [y[:, j] for j in range(packs)], axis=1
        )

    return pl.pallas_call(
        detile_kernel,
        grid=(pl.cdiv(vocab, blk),),
        in_specs=[pl.BlockSpec((dim, blk), lambda g: (0, g))],
        out_specs=pl.BlockSpec((br, 128), lambda g: (g, 0)),
        out_shape=jax.ShapeDtypeStruct((rows_out, 128), jnp.float32),
    )


def kernel(indices, table):
    batch, hist = indices.shape
    vocab, dim = table.shape
    total = batch * hist
    zero = lax.optimization_barrier(jnp.zeros((), jnp.int32))
    one = lax.optimization_barrier(jnp.ones((), jnp.float32))
    idx1 = lax.optimization_barrier(
        (indices.astype(jnp.int32) + zero).reshape(total)
    )
    det = _build_detile(vocab, dim, 25600)(table.T)
    out = _build(batch, hist, vocab, dim)(idx1, det.reshape(vocab, dim))
    out1 = lax.optimization_barrier(out.reshape(total * dim))
    return (out1 * one).reshape(batch, hist, dim)
